# baseline (speedup 1.0000x reference)
import jax
import jax.numpy as jnp
from jax import lax
from jax.experimental import pallas as pl
from jax.experimental.pallas import tpu as pltpu
from jax.experimental.pallas import tpu_sc as plsc

N = 10000
E = 320000
D = 128
H = 256
G = 128

NC = 2
NS = 16
NW = NC * NS

CHUNK = 128
NCHUNKS = -(-E // CHUNK)
NCHUNKS_PAD = -(-NCHUNKS // NW) * NW
EPAD = NCHUNKS_PAD * CHUNK
CHUNKS_PER_WORKER = NCHUNKS_PAD // NW
CHUNKS_PER_SUB = NCHUNKS_PAD // NS
L1_C0 = 68
L1_C1 = CHUNKS_PER_SUB - L1_C0

NP = 10240
ROWS_PER_SUB = NP // NS

BN = 1024
NBLK = NP // BN


def _make_sc_agg_cnt():
    out_type = [jax.ShapeDtypeStruct((NC, NP, D), jnp.float32),
                jax.ShapeDtypeStruct((NC, NP, D), jnp.float32)]
    scratch = [
        pltpu.VMEM((CHUNK,), jnp.int32),
        pltpu.VMEM((CHUNK,), jnp.int32),
        pltpu.VMEM((CHUNK, D), jnp.float32),
        pltpu.VMEM_SHARED((NP, D), jnp.float32),
        pltpu.SemaphoreType.DMA,
    ]
    mesh = plsc.VectorSubcoreMesh(
        core_axis_name="c", subcore_axis_name="s", num_cores=NC, num_subcores=NS
    )

    def body(table, srcr, dstr, zbig, ones_h, agg_out, cnt_out,
             idx_s, idx_d, rows, acc, sem):
        c = lax.axis_index("c")
        s = lax.axis_index("s")
        row0 = s * ROWS_PER_SUB
        pltpu.sync_copy(zbig.at[pl.ds(row0, ROWS_PER_SUB)],
                        acc.at[pl.ds(row0, ROWS_PER_SUB)])
        plsc.subcore_barrier()

        start = s * CHUNKS_PER_SUB + c * L1_C0
        n_chunks = jnp.where(c == 0, L1_C0, L1_C1)

        def chunk_body(j, carry):
            ch = start + j
            pltpu.sync_copy(srcr.at[ch], idx_s)
            pltpu.sync_copy(dstr.at[ch], idx_d)
            pltpu.async_copy(table.at[idx_s], rows, sem).wait()
            pltpu.sync_copy(rows, acc.at[idx_d], add=True)
            return carry

        lax.fori_loop(0, n_chunks, chunk_body, 0)
        plsc.subcore_barrier()

        pltpu.sync_copy(acc.at[pl.ds(row0, ROWS_PER_SUB)],
                        agg_out.at[c, pl.ds(row0, ROWS_PER_SUB)])

        plsc.subcore_barrier()
        pltpu.sync_copy(zbig.at[pl.ds(row0, ROWS_PER_SUB)],
                        acc.at[pl.ds(row0, ROWS_PER_SUB)])
        pltpu.sync_copy(ones_h, rows)
        plsc.subcore_barrier()

        def cnt_body(j, carry):
            pltpu.sync_copy(dstr.at[start + j], idx_d)
            pltpu.sync_copy(rows, acc.at[idx_d], add=True)
            return carry

        lax.fori_loop(0, n_chunks, cnt_body, 0)
        plsc.subcore_barrier()
        pltpu.sync_copy(acc.at[pl.ds(row0, ROWS_PER_SUB)],
                        cnt_out.at[c, pl.ds(row0, ROWS_PER_SUB)])

    return pl.kernel(body, out_type=out_type, mesh=mesh,
                     scratch_types=scratch)


def _make_sc_agg2():
    out_type = [jax.ShapeDtypeStruct((NC, NP, D), jnp.float32)]
    scratch = [
        pltpu.VMEM((CHUNK,), jnp.int32),
        pltpu.VMEM((CHUNK,), jnp.int32),
        pltpu.VMEM((CHUNK, D), jnp.float32),
        pltpu.VMEM_SHARED((NP, D), jnp.float32),
        pltpu.SemaphoreType.DMA,
    ]
    mesh = plsc.VectorSubcoreMesh(
        core_axis_name="c", subcore_axis_name="s", num_cores=NC, num_subcores=NS
    )

    def body(table, srcr, dstr, zbig, agg_out, idx_s, idx_d, rows, acc, sem):
        c = lax.axis_index("c")
        s = lax.axis_index("s")
        row0 = s * ROWS_PER_SUB
        pltpu.sync_copy(zbig.at[pl.ds(row0, ROWS_PER_SUB)],
                        acc.at[pl.ds(row0, ROWS_PER_SUB)])
        plsc.subcore_barrier()

        start = s * CHUNKS_PER_SUB
        off = c * NP

        def chunk_body(j, carry):
            ch = start + j
            pltpu.sync_copy(srcr.at[ch], idx_s)
            pltpu.sync_copy(dstr.at[ch], idx_d)
            for k in range(CHUNK // 16):
                sl = pl.ds(k * 16, 16)
                idx_s[sl] = idx_s[sl] + off
            pltpu.async_copy(table.at[idx_s], rows, sem).wait()
            pltpu.sync_copy(rows, acc.at[idx_d], add=True)
            return carry

        lax.fori_loop(0, CHUNKS_PER_SUB, chunk_body, 0)
        plsc.subcore_barrier()
        pltpu.sync_copy(acc.at[pl.ds(row0, ROWS_PER_SUB)],
                        agg_out.at[c, pl.ds(row0, ROWS_PER_SUB)])

    return pl.kernel(body, out_type=out_type, mesh=mesh,
                     scratch_types=scratch)


_sc_agg_cnt = _make_sc_agg_cnt()
_sc_agg2 = _make_sc_agg2()


def _tc_layer1_body(p_ref, c_ref, x_ref, wl_ref, wr_ref, b_ref, h_ref):
    cnt = jnp.maximum(c_ref[0, :, 0:1] + c_ref[1, :, 0:1], 1.0)
    mean = (p_ref[0] + p_ref[1]) / cnt
    acc = jnp.dot(mean, wl_ref[...], preferred_element_type=jnp.float32)
    acc = acc + jnp.dot(x_ref[...], wr_ref[...],
                        preferred_element_type=jnp.float32)
    acc = acc + b_ref[...]
    h_ref[0] = jnp.maximum(acc, 0.0)


_tc_layer1 = pl.pallas_call(
    _tc_layer1_body,
    grid=(NBLK, 2),
    in_specs=[
        pl.BlockSpec((NC, BN, D), lambda i, j: (0, i, 0)),
        pl.BlockSpec((NC, BN, D), lambda i, j: (0, i, 0)),
        pl.BlockSpec((BN, D), lambda i, j: (i, 0)),
        pl.BlockSpec((D, D), lambda i, j: (0, j)),
        pl.BlockSpec((D, D), lambda i, j: (0, j)),
        pl.BlockSpec((1, D), lambda i, j: (0, j)),
    ],
    out_specs=pl.BlockSpec((1, BN, D), lambda i, j: (j, i, 0)),
    out_shape=jax.ShapeDtypeStruct((2, NP, D), jnp.float32),
)


def _tc_layer2_body(q_ref, c_ref, h_ref, wl_ref, wr_ref, b_ref,
                    bat_ref, wo_ref, bo_ref, out_ref, pacc, gacc):
    i = pl.program_id(0)
    cnt = jnp.maximum(c_ref[0, :, 0:1] + c_ref[1, :, 0:1], 1.0)
    mean = jnp.concatenate([q_ref[0] / cnt, q_ref[1] / cnt], axis=1)
    hcat = jnp.concatenate([h_ref[0], h_ref[1]], axis=1)
    acc = jnp.dot(mean, wl_ref[...], preferred_element_type=jnp.float32)
    acc = acc + jnp.dot(hcat, wr_ref[...], preferred_element_type=jnp.float32)
    h2 = jnp.maximum(acc + b_ref[...], 0.0)

    onehot = (bat_ref[...] == lax.broadcasted_iota(jnp.int32, (BN, G), 1)
              ).astype(jnp.float32)
    psum = lax.dot_general(onehot, h2, (((0,), (0,)), ((), ())),
                           preferred_element_type=jnp.float32)
    gsum = lax.dot_general(onehot, jnp.ones((BN, 8), jnp.float32),
                           (((0,), (0,)), ((), ())),
                           preferred_element_type=jnp.float32)

    @pl.when(i == 0)
    def _():
        pacc[...] = psum
        gacc[...] = gsum

    @pl.when(i > 0)
    def _():
        pacc[...] += psum
        gacc[...] += gsum

    @pl.when(i == NBLK - 1)
    def _():
        gc = jnp.maximum(gacc[:, 0:1], 1.0)
        out_ref[...] = (jnp.dot(pacc[...], wo_ref[...],
                                preferred_element_type=jnp.float32) / gc
                        + bo_ref[...])


_tc_layer2 = pl.pallas_call(
    _tc_layer2_body,
    grid=(NBLK,),
    in_specs=[
        pl.BlockSpec((NC, BN, D), lambda i: (0, i, 0)),
        pl.BlockSpec((NC, BN, D), lambda i: (0, i, 0)),
        pl.BlockSpec((NC, BN, D), lambda i: (0, i, 0)),
        pl.BlockSpec((H, H), lambda i: (0, 0)),
        pl.BlockSpec((H, H), lambda i: (0, 0)),
        pl.BlockSpec((1, H), lambda i: (0, 0)),
        pl.BlockSpec((BN, 1), lambda i: (i, 0)),
        pl.BlockSpec((H, G), lambda i: (0, 0)),
        pl.BlockSpec((1, G), lambda i: (0, 0)),
    ],
    out_specs=pl.BlockSpec((G, G), lambda i: (0, 0)),
    out_shape=jax.ShapeDtypeStruct((G, G), jnp.float32),
    scratch_shapes=[
        pltpu.VMEM((G, H), jnp.float32),
        pltpu.VMEM((G, 8), jnp.float32),
    ],
)


def kernel(x, edge_index, batch, W_l1, W_r1, b1, W_l2, W_r2, b2, W_out, b_out):
    src = edge_index[0]
    dst = edge_index[1]
    pad_e = EPAD - E
    srcp = jnp.concatenate(
        [src, jnp.full((pad_e,), N, jnp.int32)]).reshape(NCHUNKS_PAD, CHUNK)
    dstp = jnp.concatenate(
        [dst, jnp.full((pad_e,), N, jnp.int32)]).reshape(NCHUNKS_PAD, CHUNK)
    xp = jnp.pad(x, ((0, NP - N), (0, 0)))
    zbig = jnp.zeros((NP, D), jnp.float32)
    ones_h = jnp.ones((CHUNK, D), jnp.float32)
    batp = jnp.pad(batch, (0, NP - N), constant_values=G).reshape(NP, 1)
    wo_pad = jnp.zeros((H, G), jnp.float32).at[:, :2].set(W_out)
    bo_pad = jnp.zeros((1, G), jnp.float32).at[:, :2].set(b_out)

    p1, c1 = _sc_agg_cnt(xp, srcp, dstp, zbig, ones_h)
    h = _tc_layer1(p1, c1, xp, W_l1, W_r1, b1.reshape(1, H))
    (q,) = _sc_agg2(h.reshape(NC * NP, D), srcp, dstp, zbig)
    out = _tc_layer2(q, c1, h, W_l2, W_r2, b2.reshape(1, H),
                     batp, wo_pad, bo_pad)
    return out[:, :2]


# speedup vs baseline: 1.0697x; 1.0697x over previous
import jax
import jax.numpy as jnp
from jax import lax
from jax.experimental import pallas as pl
from jax.experimental.pallas import tpu as pltpu
from jax.experimental.pallas import tpu_sc as plsc

N = 10000
E = 320000
D = 128
H = 256
G = 128

NC = 2
NS = 16
NW = NC * NS

CHUNK = 128
NCHUNKS = -(-E // CHUNK)
NCHUNKS_PAD = -(-NCHUNKS // NW) * NW
EPAD = NCHUNKS_PAD * CHUNK
CHUNKS_PER_WORKER = NCHUNKS_PAD // NW
CHUNKS_PER_SUB = NCHUNKS_PAD // NS
L1_C0 = 90
L1_C1 = CHUNKS_PER_SUB - L1_C0

NP = 10240
ROWS_PER_SUB = NP // NS

BN = 1024
NBLK = NP // BN


def _make_sc_agg_cnt():
    out_type = [jax.ShapeDtypeStruct((NC, NP, D), jnp.float32),
                jax.ShapeDtypeStruct((NC, NP, D), jnp.float32)]
    scratch = [
        pltpu.VMEM((CHUNK,), jnp.int32),
        pltpu.VMEM((CHUNK,), jnp.int32),
        pltpu.VMEM((CHUNK, D), jnp.float32),
        pltpu.VMEM_SHARED((NP, D), jnp.float32),
        pltpu.SemaphoreType.DMA,
    ]
    mesh = plsc.VectorSubcoreMesh(
        core_axis_name="c", subcore_axis_name="s", num_cores=NC, num_subcores=NS
    )

    def body(table, srcr, dstr, zbig, ones_h, agg_out, cnt_out,
             idx_s, idx_d, rows, acc, sem):
        c = lax.axis_index("c")
        s = lax.axis_index("s")
        row0 = s * ROWS_PER_SUB
        pltpu.sync_copy(zbig.at[pl.ds(row0, ROWS_PER_SUB)],
                        acc.at[pl.ds(row0, ROWS_PER_SUB)])
        plsc.subcore_barrier()

        start = s * CHUNKS_PER_SUB + c * L1_C0
        n_chunks = jnp.where(c == 0, L1_C0, L1_C1)

        def chunk_body(j, carry):
            ch = start + j
            pltpu.sync_copy(srcr.at[ch], idx_s)
            pltpu.sync_copy(dstr.at[ch], idx_d)
            pltpu.async_copy(table.at[idx_s], rows, sem).wait()
            pltpu.sync_copy(rows, acc.at[idx_d], add=True)
            return carry

        lax.fori_loop(0, n_chunks, chunk_body, 0)
        plsc.subcore_barrier()

        pltpu.sync_copy(acc.at[pl.ds(row0, ROWS_PER_SUB)],
                        agg_out.at[c, pl.ds(row0, ROWS_PER_SUB)])

        plsc.subcore_barrier()
        pltpu.sync_copy(zbig.at[pl.ds(row0, ROWS_PER_SUB)],
                        acc.at[pl.ds(row0, ROWS_PER_SUB)])
        pltpu.sync_copy(ones_h, rows)
        plsc.subcore_barrier()

        def cnt_body(j, carry):
            pltpu.sync_copy(dstr.at[start + j], idx_d)
            pltpu.sync_copy(rows, acc.at[idx_d], add=True)
            return carry

        lax.fori_loop(0, n_chunks, cnt_body, 0)
        plsc.subcore_barrier()
        pltpu.sync_copy(acc.at[pl.ds(row0, ROWS_PER_SUB)],
                        cnt_out.at[c, pl.ds(row0, ROWS_PER_SUB)])

    return pl.kernel(body, out_type=out_type, mesh=mesh,
                     scratch_types=scratch)


def _make_sc_agg2():
    out_type = [jax.ShapeDtypeStruct((NC, NP, D), jnp.float32)]
    scratch = [
        pltpu.VMEM((CHUNK,), jnp.int32),
        pltpu.VMEM((CHUNK,), jnp.int32),
        pltpu.VMEM((CHUNK, D), jnp.float32),
        pltpu.VMEM_SHARED((NP, D), jnp.float32),
        pltpu.SemaphoreType.DMA,
    ]
    mesh = plsc.VectorSubcoreMesh(
        core_axis_name="c", subcore_axis_name="s", num_cores=NC, num_subcores=NS
    )

    def body(table, srcr, dstr, zbig, agg_out, idx_s, idx_d, rows, acc, sem):
        c = lax.axis_index("c")
        s = lax.axis_index("s")
        row0 = s * ROWS_PER_SUB
        pltpu.sync_copy(zbig.at[pl.ds(row0, ROWS_PER_SUB)],
                        acc.at[pl.ds(row0, ROWS_PER_SUB)])
        plsc.subcore_barrier()

        start = s * CHUNKS_PER_SUB
        off = c * NP

        def chunk_body(j, carry):
            ch = start + j
            pltpu.sync_copy(srcr.at[ch], idx_s)
            pltpu.sync_copy(dstr.at[ch], idx_d)
            for k in range(CHUNK // 16):
                sl = pl.ds(k * 16, 16)
                idx_s[sl] = idx_s[sl] + off
            pltpu.async_copy(table.at[idx_s], rows, sem).wait()
            pltpu.sync_copy(rows, acc.at[idx_d], add=True)
            return carry

        lax.fori_loop(0, CHUNKS_PER_SUB, chunk_body, 0)
        plsc.subcore_barrier()
        pltpu.sync_copy(acc.at[pl.ds(row0, ROWS_PER_SUB)],
                        agg_out.at[c, pl.ds(row0, ROWS_PER_SUB)])

    return pl.kernel(body, out_type=out_type, mesh=mesh,
                     scratch_types=scratch)


_sc_agg_cnt = _make_sc_agg_cnt()
_sc_agg2 = _make_sc_agg2()


def _tc_layer1_body(p_ref, c_ref, x_ref, wl_ref, wr_ref, b_ref, h_ref):
    cnt = jnp.maximum(c_ref[0, :, 0:1] + c_ref[1, :, 0:1], 1.0)
    mean = (p_ref[0] + p_ref[1]) / cnt
    acc = jnp.dot(mean, wl_ref[...], preferred_element_type=jnp.float32)
    acc = acc + jnp.dot(x_ref[...], wr_ref[...],
                        preferred_element_type=jnp.float32)
    acc = acc + b_ref[...]
    h_ref[0] = jnp.maximum(acc, 0.0)


_tc_layer1 = pl.pallas_call(
    _tc_layer1_body,
    grid=(NBLK, 2),
    in_specs=[
        pl.BlockSpec((NC, BN, D), lambda i, j: (0, i, 0)),
        pl.BlockSpec((NC, BN, D), lambda i, j: (0, i, 0)),
        pl.BlockSpec((BN, D), lambda i, j: (i, 0)),
        pl.BlockSpec((D, D), lambda i, j: (0, j)),
        pl.BlockSpec((D, D), lambda i, j: (0, j)),
        pl.BlockSpec((1, D), lambda i, j: (0, j)),
    ],
    out_specs=pl.BlockSpec((1, BN, D), lambda i, j: (j, i, 0)),
    out_shape=jax.ShapeDtypeStruct((2, NP, D), jnp.float32),
)


def _tc_layer2_body(q_ref, c_ref, h_ref, wl_ref, wr_ref, b_ref,
                    bat_ref, wo_ref, bo_ref, out_ref, pacc, gacc):
    i = pl.program_id(0)
    cnt = jnp.maximum(c_ref[0, :, 0:1] + c_ref[1, :, 0:1], 1.0)
    mean = jnp.concatenate([q_ref[0] / cnt, q_ref[1] / cnt], axis=1)
    hcat = jnp.concatenate([h_ref[0], h_ref[1]], axis=1)
    acc = jnp.dot(mean, wl_ref[...], preferred_element_type=jnp.float32)
    acc = acc + jnp.dot(hcat, wr_ref[...], preferred_element_type=jnp.float32)
    h2 = jnp.maximum(acc + b_ref[...], 0.0)

    onehot = (bat_ref[...] == lax.broadcasted_iota(jnp.int32, (BN, G), 1)
              ).astype(jnp.float32)
    psum = lax.dot_general(onehot, h2, (((0,), (0,)), ((), ())),
                           preferred_element_type=jnp.float32)
    gsum = lax.dot_general(onehot, jnp.ones((BN, 8), jnp.float32),
                           (((0,), (0,)), ((), ())),
                           preferred_element_type=jnp.float32)

    @pl.when(i == 0)
    def _():
        pacc[...] = psum
        gacc[...] = gsum

    @pl.when(i > 0)
    def _():
        pacc[...] += psum
        gacc[...] += gsum

    @pl.when(i == NBLK - 1)
    def _():
        gc = jnp.maximum(gacc[:, 0:1], 1.0)
        out_ref[...] = (jnp.dot(pacc[...], wo_ref[...],
                                preferred_element_type=jnp.float32) / gc
                        + bo_ref[...])


_tc_layer2 = pl.pallas_call(
    _tc_layer2_body,
    grid=(NBLK,),
    in_specs=[
        pl.BlockSpec((NC, BN, D), lambda i: (0, i, 0)),
        pl.BlockSpec((NC, BN, D), lambda i: (0, i, 0)),
        pl.BlockSpec((NC, BN, D), lambda i: (0, i, 0)),
        pl.BlockSpec((H, H), lambda i: (0, 0)),
        pl.BlockSpec((H, H), lambda i: (0, 0)),
        pl.BlockSpec((1, H), lambda i: (0, 0)),
        pl.BlockSpec((BN, 1), lambda i: (i, 0)),
        pl.BlockSpec((H, G), lambda i: (0, 0)),
        pl.BlockSpec((1, G), lambda i: (0, 0)),
    ],
    out_specs=pl.BlockSpec((G, G), lambda i: (0, 0)),
    out_shape=jax.ShapeDtypeStruct((G, G), jnp.float32),
    scratch_shapes=[
        pltpu.VMEM((G, H), jnp.float32),
        pltpu.VMEM((G, 8), jnp.float32),
    ],
)


def kernel(x, edge_index, batch, W_l1, W_r1, b1, W_l2, W_r2, b2, W_out, b_out):
    src = edge_index[0]
    dst = edge_index[1]
    pad_e = EPAD - E
    srcp = jnp.concatenate(
        [src, jnp.full((pad_e,), N, jnp.int32)]).reshape(NCHUNKS_PAD, CHUNK)
    dstp = jnp.concatenate(
        [dst, jnp.full((pad_e,), N, jnp.int32)]).reshape(NCHUNKS_PAD, CHUNK)
    xp = jnp.pad(x, ((0, NP - N), (0, 0)))
    zbig = jnp.zeros((NP, D), jnp.float32)
    ones_h = jnp.ones((CHUNK, D), jnp.float32)
    batp = jnp.pad(batch, (0, NP - N), constant_values=G).reshape(NP, 1)
    wo_pad = jnp.zeros((H, G), jnp.float32).at[:, :2].set(W_out)
    bo_pad = jnp.zeros((1, G), jnp.float32).at[:, :2].set(b_out)

    p1, c1 = _sc_agg_cnt(xp, srcp, dstp, zbig, ones_h)
    h = _tc_layer1(p1, c1, xp, W_l1, W_r1, b1.reshape(1, H))
    (q,) = _sc_agg2(h.reshape(NC * NP, D), srcp, dstp, zbig)
    out = _tc_layer2(q, c1, h, W_l2, W_r2, b2.reshape(1, H),
                     batp, wo_pad, bo_pad)
    return out[:, :2]


# speedup vs baseline: 1.0847x; 1.0140x over previous
import jax
import jax.numpy as jnp
from jax import lax
from jax.experimental import pallas as pl
from jax.experimental.pallas import tpu as pltpu
from jax.experimental.pallas import tpu_sc as plsc

N = 10000
E = 320000
D = 128
H = 256
G = 128

NC = 2
NS = 16
NW = NC * NS

CHUNK = 128
NCHUNKS = -(-E // CHUNK)
NCHUNKS_PAD = -(-NCHUNKS // NW) * NW
EPAD = NCHUNKS_PAD * CHUNK
CHUNKS_PER_WORKER = NCHUNKS_PAD // NW
CHUNKS_PER_SUB = NCHUNKS_PAD // NS
L1_C0 = 94
L1_C1 = CHUNKS_PER_SUB - L1_C0

NP = 10240
ROWS_PER_SUB = NP // NS

BN = 1024
NBLK = NP // BN


def _make_sc_agg_cnt():
    out_type = [jax.ShapeDtypeStruct((NC, NP, D), jnp.float32),
                jax.ShapeDtypeStruct((NC, NP, D), jnp.float32)]
    scratch = [
        pltpu.VMEM((CHUNK,), jnp.int32),
        pltpu.VMEM((CHUNK,), jnp.int32),
        pltpu.VMEM((CHUNK, D), jnp.float32),
        pltpu.VMEM_SHARED((NP, D), jnp.float32),
        pltpu.SemaphoreType.DMA,
    ]
    mesh = plsc.VectorSubcoreMesh(
        core_axis_name="c", subcore_axis_name="s", num_cores=NC, num_subcores=NS
    )

    def body(table, srcr, dstr, zbig, ones_h, agg_out, cnt_out,
             idx_s, idx_d, rows, acc, sem):
        c = lax.axis_index("c")
        s = lax.axis_index("s")
        row0 = s * ROWS_PER_SUB
        pltpu.sync_copy(zbig.at[pl.ds(row0, ROWS_PER_SUB)],
                        acc.at[pl.ds(row0, ROWS_PER_SUB)])
        plsc.subcore_barrier()

        start = s * CHUNKS_PER_SUB + c * L1_C0
        n_chunks = jnp.where(c == 0, L1_C0, L1_C1)

        def chunk_body(j, carry):
            ch = start + j
            pltpu.sync_copy(srcr.at[ch], idx_s)
            pltpu.sync_copy(dstr.at[ch], idx_d)
            pltpu.async_copy(table.at[idx_s], rows, sem).wait()
            pltpu.sync_copy(rows, acc.at[idx_d], add=True)
            return carry

        lax.fori_loop(0, n_chunks, chunk_body, 0)
        plsc.subcore_barrier()

        pltpu.sync_copy(acc.at[pl.ds(row0, ROWS_PER_SUB)],
                        agg_out.at[c, pl.ds(row0, ROWS_PER_SUB)])

        plsc.subcore_barrier()
        pltpu.sync_copy(zbig.at[pl.ds(row0, ROWS_PER_SUB)],
                        acc.at[pl.ds(row0, ROWS_PER_SUB)])
        pltpu.sync_copy(ones_h, rows)
        plsc.subcore_barrier()

        def cnt_body(j, carry):
            pltpu.sync_copy(dstr.at[start + j], idx_d)
            pltpu.sync_copy(rows, acc.at[idx_d], add=True)
            return carry

        lax.fori_loop(0, n_chunks, cnt_body, 0)
        plsc.subcore_barrier()
        pltpu.sync_copy(acc.at[pl.ds(row0, ROWS_PER_SUB)],
                        cnt_out.at[c, pl.ds(row0, ROWS_PER_SUB)])

    return pl.kernel(body, out_type=out_type, mesh=mesh,
                     scratch_types=scratch)


def _make_sc_agg2():
    out_type = [jax.ShapeDtypeStruct((NC, NP, D), jnp.float32)]
    scratch = [
        pltpu.VMEM((CHUNK,), jnp.int32),
        pltpu.VMEM((CHUNK,), jnp.int32),
        pltpu.VMEM((CHUNK, D), jnp.float32),
        pltpu.VMEM_SHARED((NP, D), jnp.float32),
        pltpu.SemaphoreType.DMA,
    ]
    mesh = plsc.VectorSubcoreMesh(
        core_axis_name="c", subcore_axis_name="s", num_cores=NC, num_subcores=NS
    )

    def body(table, srcr, dstr, zbig, agg_out, idx_s, idx_d, rows, acc, sem):
        c = lax.axis_index("c")
        s = lax.axis_index("s")
        row0 = s * ROWS_PER_SUB
        pltpu.sync_copy(zbig.at[pl.ds(row0, ROWS_PER_SUB)],
                        acc.at[pl.ds(row0, ROWS_PER_SUB)])
        plsc.subcore_barrier()

        start = s * CHUNKS_PER_SUB
        off = c * NP

        def chunk_body(j, carry):
            ch = start + j
            pltpu.sync_copy(srcr.at[ch], idx_s)
            pltpu.sync_copy(dstr.at[ch], idx_d)
            for k in range(CHUNK // 16):
                sl = pl.ds(k * 16, 16)
                idx_s[sl] = idx_s[sl] + off
            pltpu.async_copy(table.at[idx_s], rows, sem).wait()
            pltpu.sync_copy(rows, acc.at[idx_d], add=True)
            return carry

        lax.fori_loop(0, CHUNKS_PER_SUB, chunk_body, 0)
        plsc.subcore_barrier()
        pltpu.sync_copy(acc.at[pl.ds(row0, ROWS_PER_SUB)],
                        agg_out.at[c, pl.ds(row0, ROWS_PER_SUB)])

    return pl.kernel(body, out_type=out_type, mesh=mesh,
                     scratch_types=scratch)


_sc_agg_cnt = _make_sc_agg_cnt()
_sc_agg2 = _make_sc_agg2()


def _tc_layer1_body(p_ref, c_ref, x_ref, wl_ref, wr_ref, b_ref, h_ref):
    cnt = jnp.maximum(c_ref[0, :, 0:1] + c_ref[1, :, 0:1], 1.0)
    mean = (p_ref[0] + p_ref[1]) / cnt
    acc = jnp.dot(mean, wl_ref[...], preferred_element_type=jnp.float32)
    acc = acc + jnp.dot(x_ref[...], wr_ref[...],
                        preferred_element_type=jnp.float32)
    acc = acc + b_ref[...]
    h_ref[0] = jnp.maximum(acc, 0.0)


_tc_layer1 = pl.pallas_call(
    _tc_layer1_body,
    grid=(NBLK, 2),
    in_specs=[
        pl.BlockSpec((NC, BN, D), lambda i, j: (0, i, 0)),
        pl.BlockSpec((NC, BN, D), lambda i, j: (0, i, 0)),
        pl.BlockSpec((BN, D), lambda i, j: (i, 0)),
        pl.BlockSpec((D, D), lambda i, j: (0, j)),
        pl.BlockSpec((D, D), lambda i, j: (0, j)),
        pl.BlockSpec((1, D), lambda i, j: (0, j)),
    ],
    out_specs=pl.BlockSpec((1, BN, D), lambda i, j: (j, i, 0)),
    out_shape=jax.ShapeDtypeStruct((2, NP, D), jnp.float32),
)


def _tc_layer2_body(q_ref, c_ref, h_ref, wl_ref, wr_ref, b_ref,
                    bat_ref, wo_ref, bo_ref, out_ref, pacc, gacc):
    i = pl.program_id(0)
    cnt = jnp.maximum(c_ref[0, :, 0:1] + c_ref[1, :, 0:1], 1.0)
    mean = jnp.concatenate([q_ref[0] / cnt, q_ref[1] / cnt], axis=1)
    hcat = jnp.concatenate([h_ref[0], h_ref[1]], axis=1)
    acc = jnp.dot(mean, wl_ref[...], preferred_element_type=jnp.float32)
    acc = acc + jnp.dot(hcat, wr_ref[...], preferred_element_type=jnp.float32)
    h2 = jnp.maximum(acc + b_ref[...], 0.0)

    onehot = (bat_ref[...] == lax.broadcasted_iota(jnp.int32, (BN, G), 1)
              ).astype(jnp.float32)
    psum = lax.dot_general(onehot, h2, (((0,), (0,)), ((), ())),
                           preferred_element_type=jnp.float32)
    gsum = lax.dot_general(onehot, jnp.ones((BN, 8), jnp.float32),
                           (((0,), (0,)), ((), ())),
                           preferred_element_type=jnp.float32)

    @pl.when(i == 0)
    def _():
        pacc[...] = psum
        gacc[...] = gsum

    @pl.when(i > 0)
    def _():
        pacc[...] += psum
        gacc[...] += gsum

    @pl.when(i == NBLK - 1)
    def _():
        gc = jnp.maximum(gacc[:, 0:1], 1.0)
        out_ref[...] = (jnp.dot(pacc[...], wo_ref[...],
                                preferred_element_type=jnp.float32) / gc
                        + bo_ref[...])


_tc_layer2 = pl.pallas_call(
    _tc_layer2_body,
    grid=(NBLK,),
    in_specs=[
        pl.BlockSpec((NC, BN, D), lambda i: (0, i, 0)),
        pl.BlockSpec((NC, BN, D), lambda i: (0, i, 0)),
        pl.BlockSpec((NC, BN, D), lambda i: (0, i, 0)),
        pl.BlockSpec((H, H), lambda i: (0, 0)),
        pl.BlockSpec((H, H), lambda i: (0, 0)),
        pl.BlockSpec((1, H), lambda i: (0, 0)),
        pl.BlockSpec((BN, 1), lambda i: (i, 0)),
        pl.BlockSpec((H, G), lambda i: (0, 0)),
        pl.BlockSpec((1, G), lambda i: (0, 0)),
    ],
    out_specs=pl.BlockSpec((G, G), lambda i: (0, 0)),
    out_shape=jax.ShapeDtypeStruct((G, G), jnp.float32),
    scratch_shapes=[
        pltpu.VMEM((G, H), jnp.float32),
        pltpu.VMEM((G, 8), jnp.float32),
    ],
)


def kernel(x, edge_index, batch, W_l1, W_r1, b1, W_l2, W_r2, b2, W_out, b_out):
    src = edge_index[0]
    dst = edge_index[1]
    pad_e = EPAD - E
    srcp = jnp.concatenate(
        [src, jnp.full((pad_e,), N, jnp.int32)]).reshape(NCHUNKS_PAD, CHUNK)
    dstp = jnp.concatenate(
        [dst, jnp.full((pad_e,), N, jnp.int32)]).reshape(NCHUNKS_PAD, CHUNK)
    xp = jnp.pad(x, ((0, NP - N), (0, 0)))
    zbig = jnp.zeros((NP, D), jnp.float32)
    ones_h = jnp.ones((CHUNK, D), jnp.float32)
    batp = jnp.pad(batch, (0, NP - N), constant_values=G).reshape(NP, 1)
    wo_pad = jnp.zeros((H, G), jnp.float32).at[:, :2].set(W_out)
    bo_pad = jnp.zeros((1, G), jnp.float32).at[:, :2].set(b_out)

    p1, c1 = _sc_agg_cnt(xp, srcp, dstp, zbig, ones_h)
    h = _tc_layer1(p1, c1, xp, W_l1, W_r1, b1.reshape(1, H))
    (q,) = _sc_agg2(h.reshape(NC * NP, D), srcp, dstp, zbig)
    out = _tc_layer2(q, c1, h, W_l2, W_r2, b2.reshape(1, H),
                     batp, wo_pad, bo_pad)
    return out[:, :2]
